# baseline (speedup 1.0000x reference)
import functools

import jax
import jax.numpy as jnp
from jax import lax
from jax.experimental import pallas as pl
from jax.experimental.pallas import tpu as pltpu
from jax.experimental.pallas import tpu_sc as plsc

NUM_EXPERTS = 64
TOP_K = 2
D_MODEL = 768
SEQ = 2048
SLOTS = SEQ * TOP_K
BLK = 64
OFF_PAD = 80
LANES = 16
NCHUNK = SLOTS // LANES
SLOTS_P = SLOTS + NUM_EXPERTS * 8


def _routing_body(ei_hbm, ew_hbm, stok_hbm, sew_hbm, off_hbm, cnt_hbm,
                  ei_v, ew_v, stok_v, sew_v, off_v, cnt_v, cur_v):
    c = lax.axis_index("c")
    s = lax.axis_index("s")

    @pl.when(jnp.logical_and(c == 0, s == 0))
    def _():
        pltpu.sync_copy(ei_hbm, ei_v)
        pltpu.sync_copy(ew_hbm, ew_v)

        d0, _unused = plsc.scan_count(jnp.zeros((LANES,), jnp.int32))
        cbase = d0[0]

        zeros16 = jnp.zeros((LANES,), jnp.int32)
        for k in range(NUM_EXPERTS // LANES):
            cnt_v[pl.ds(k * LANES, LANES)] = zeros16

        def hist(i, _):
            ids = ei_v[pl.ds(i * LANES, LANES)]
            dup, last = plsc.scan_count(ids)
            plsc.addupdate_scatter(cnt_v, [ids], dup - cbase + 1, mask=last)
            return 0
        lax.fori_loop(0, NCHUNK, hist, 0)

        iota = lax.iota(jnp.int32, LANES)
        carry = jnp.int32(0)
        for k in range(NUM_EXPERTS // LANES):
            cnt = cnt_v[pl.ds(k * LANES, LANES)]
            pcnt = jnp.bitwise_and(cnt + 7, ~jnp.int32(7))
            inc = plsc.cumsum(pcnt)
            excl = inc - pcnt + carry
            cur_v[pl.ds(k * LANES, LANES)] = excl
            off_v[pl.ds(k * LANES, LANES)] = excl
            carry = carry + inc[LANES - 1]
        off_v[pl.ds(NUM_EXPERTS, LANES)] = jnp.where(iota == 0, carry, 0)

        zf = jnp.zeros((LANES,), jnp.float32)

        def zbuf(k, _):
            stok_v[pl.ds(k * LANES, LANES)] = zeros16
            sew_v[pl.ds(k * LANES, LANES)] = zf
            return 0
        lax.fori_loop(0, SLOTS_P // LANES, zbuf, 0)

        def place(i, _):
            base = i * LANES
            ids = ei_v[pl.ds(base, LANES)]
            ewv = ew_v[pl.ds(base, LANES)]
            toks = lax.shift_right_logical(base + iota, 1)
            dup, last = plsc.scan_count(ids)
            cur = plsc.load_gather(cur_v, [ids])
            pos = cur + (dup - cbase)
            plsc.store_scatter(stok_v, [pos], toks)
            plsc.store_scatter(sew_v, [pos], ewv)
            plsc.store_scatter(cur_v, [ids], pos + 1, mask=last)
            return 0
        lax.fori_loop(0, NCHUNK, place, 0)

        pltpu.sync_copy(stok_v, stok_hbm)
        pltpu.sync_copy(sew_v, sew_hbm)
        pltpu.sync_copy(off_v, off_hbm)
        pltpu.sync_copy(cnt_v, cnt_hbm)


def _sc_routing(ei, ew):
    mesh = plsc.VectorSubcoreMesh(core_axis_name="c", subcore_axis_name="s")
    fn = pl.kernel(
        _routing_body,
        mesh=mesh,
        compiler_params=pltpu.CompilerParams(needs_layout_passes=False),
        out_type=[
            jax.ShapeDtypeStruct((SLOTS_P,), jnp.int32),
            jax.ShapeDtypeStruct((SLOTS_P,), jnp.float32),
            jax.ShapeDtypeStruct((OFF_PAD,), jnp.int32),
            jax.ShapeDtypeStruct((NUM_EXPERTS,), jnp.int32),
        ],
        scratch_types=[
            pltpu.VMEM((SLOTS,), jnp.int32),
            pltpu.VMEM((SLOTS,), jnp.float32),
            pltpu.VMEM((SLOTS_P,), jnp.int32),
            pltpu.VMEM((SLOTS_P,), jnp.float32),
            pltpu.VMEM((OFF_PAD,), jnp.int32),
            pltpu.VMEM((NUM_EXPERTS,), jnp.int32),
            pltpu.VMEM((NUM_EXPERTS,), jnp.int32),
        ],
    )
    return fn(ei, ew)


def _moe_body(tok_ref, ew_ref, off_ref, cnt_ref, xs_ref, w1_ref, w2_ref,
              y_ref, eo_ref, o_ref):
    e = pl.program_id(0)

    @pl.when(e == 0)
    def _():
        y_ref[...] = jnp.zeros_like(y_ref)

    eo_ref[...] = jnp.zeros_like(eo_ref)

    start = pl.multiple_of(off_ref[e], 8)
    end = start + cnt_ref[e]
    nblk = (end - start + BLK - 1) // BLK

    def blk_body(b, _):
        base = start + b * BLK

        xg = xs_ref[pl.ds(base, BLK), :].astype(jnp.bfloat16)
        h = jax.nn.gelu(jnp.dot(xg, w1_ref[0],
                                preferred_element_type=jnp.float32))
        o_ref[...] = jnp.dot(h.astype(jnp.bfloat16), w2_ref[0],
                             preferred_element_type=jnp.float32)

        rows = jnp.minimum(end - base, BLK)

        def scatter_row(r, _):
            slot = base + r
            tok = tok_ref[slot]
            row = o_ref[pl.ds(r, 1), :]
            y_ref[pl.ds(tok, 1), :] = y_ref[pl.ds(tok, 1), :] + row * ew_ref[slot]
            eo_ref[0, 0, pl.ds(tok, 1), :] = eo_ref[0, 0, pl.ds(tok, 1), :] + row
            return 0

        lax.fori_loop(0, rows, scatter_row, 0)
        return 0

    lax.fori_loop(0, nblk, blk_body, 0)


GCHUNK = SLOTS_P // 32


def _gather_body(stok_hbm, x_hbm, xs_hbm, idx_v, rows_v, sem):
    c = lax.axis_index("c")
    s = lax.axis_index("s")
    wid = s * 2 + c
    base = wid * GCHUNK
    pltpu.sync_copy(stok_hbm.at[pl.ds(base, GCHUNK)], idx_v)
    pltpu.async_copy(x_hbm.at[idx_v], rows_v, sem).wait()
    pltpu.sync_copy(rows_v, xs_hbm.at[pl.ds(base, GCHUNK)])


def _sc_gather(stok, xf):
    mesh = plsc.VectorSubcoreMesh(core_axis_name="c", subcore_axis_name="s")
    fn = pl.kernel(
        _gather_body,
        mesh=mesh,
        out_type=jax.ShapeDtypeStruct((SLOTS_P + BLK, D_MODEL), jnp.float32),
        scratch_types=[
            pltpu.VMEM((GCHUNK,), jnp.int32),
            pltpu.VMEM((GCHUNK, D_MODEL), jnp.float32),
            pltpu.SemaphoreType.DMA,
        ],
    )
    return fn(stok, xf)


@jax.jit
def _moe_call(sorted_tok, sorted_ew, offsets, counts, xs, w1, w2):
    grid_spec = pltpu.PrefetchScalarGridSpec(
        num_scalar_prefetch=4,
        grid=(NUM_EXPERTS,),
        in_specs=[
            pl.BlockSpec((SLOTS_P + BLK, D_MODEL), lambda e, *_: (0, 0)),
            pl.BlockSpec((1, D_MODEL, D_MODEL), lambda e, *_: (e, 0, 0)),
            pl.BlockSpec((1, D_MODEL, D_MODEL), lambda e, *_: (e, 0, 0)),
        ],
        out_specs=[
            pl.BlockSpec((SEQ, D_MODEL), lambda e, *_: (0, 0)),
            pl.BlockSpec((1, 1, SEQ, D_MODEL), lambda e, *_: (0, e, 0, 0)),
        ],
        scratch_shapes=[
            pltpu.VMEM((BLK, D_MODEL), jnp.float32),
        ],
    )
    return pl.pallas_call(
        _moe_body,
        grid_spec=grid_spec,
        out_shape=[
            jax.ShapeDtypeStruct((SEQ, D_MODEL), jnp.float32),
            jax.ShapeDtypeStruct((1, NUM_EXPERTS, SEQ, D_MODEL), jnp.float32),
        ],
    )(sorted_tok, sorted_ew, offsets, counts, xs, w1, w2)


def kernel(x, expert_weights, expert_indices, w1, w2):
    sl, bs, hs = x.shape
    xf = x.reshape(-1, hs)

    ei = expert_indices.reshape(-1).astype(jnp.int32)
    ewf = expert_weights.reshape(-1)
    sorted_tok, sorted_ew, offsets, counts = _sc_routing(ei, ewf)
    offsets = offsets[:NUM_EXPERTS + 1]
    xs = _sc_gather(sorted_tok, xf)

    y, eo = _moe_call(sorted_tok, sorted_ew, offsets, counts, xs,
                      w1.astype(jnp.bfloat16), w2.astype(jnp.bfloat16))
    return y.reshape(sl, bs, hs), eo


# speedup vs baseline: 1.2801x; 1.2801x over previous
import functools

import jax
import jax.numpy as jnp
from jax import lax
from jax.experimental import pallas as pl
from jax.experimental.pallas import tpu as pltpu
from jax.experimental.pallas import tpu_sc as plsc

NUM_EXPERTS = 64
TOP_K = 2
D_MODEL = 768
SEQ = 2048
SLOTS = SEQ * TOP_K
BLK = 64
OFF_PAD = 80
LANES = 16
NCHUNK = SLOTS // LANES
SLOTS_P = SLOTS + NUM_EXPERTS * 8


def _routing_body(ei_hbm, ew_hbm, stok_hbm, sew_hbm, off_hbm, cnt_hbm,
                  ei_v, ew_v, stok_v, sew_v, off_v, cnt_v, cur_v):
    c = lax.axis_index("c")
    s = lax.axis_index("s")

    @pl.when(jnp.logical_and(c == 0, s == 0))
    def _():
        pltpu.sync_copy(ei_hbm, ei_v)
        pltpu.sync_copy(ew_hbm, ew_v)

        d0, _unused = plsc.scan_count(jnp.zeros((LANES,), jnp.int32))
        cbase = d0[0]

        zeros16 = jnp.zeros((LANES,), jnp.int32)
        for k in range(NUM_EXPERTS // LANES):
            cnt_v[pl.ds(k * LANES, LANES)] = zeros16

        def hist(i, _):
            ids = ei_v[pl.ds(i * LANES, LANES)]
            dup, last = plsc.scan_count(ids)
            plsc.addupdate_scatter(cnt_v, [ids], dup - cbase + 1, mask=last)
            return 0
        lax.fori_loop(0, NCHUNK, hist, 0)

        iota = lax.iota(jnp.int32, LANES)
        carry = jnp.int32(0)
        for k in range(NUM_EXPERTS // LANES):
            cnt = cnt_v[pl.ds(k * LANES, LANES)]
            pcnt = jnp.bitwise_and(cnt + 7, ~jnp.int32(7))
            inc = plsc.cumsum(pcnt)
            excl = inc - pcnt + carry
            cur_v[pl.ds(k * LANES, LANES)] = excl
            off_v[pl.ds(k * LANES, LANES)] = excl
            carry = carry + inc[LANES - 1]
        off_v[pl.ds(NUM_EXPERTS, LANES)] = jnp.where(iota == 0, carry, 0)

        zf = jnp.zeros((LANES,), jnp.float32)

        def zbuf(k, _):
            stok_v[pl.ds(k * LANES, LANES)] = zeros16
            sew_v[pl.ds(k * LANES, LANES)] = zf
            return 0
        lax.fori_loop(0, SLOTS_P // LANES, zbuf, 0)

        def place(i, _):
            base = i * LANES
            ids = ei_v[pl.ds(base, LANES)]
            ewv = ew_v[pl.ds(base, LANES)]
            toks = lax.shift_right_logical(base + iota, 1)
            dup, last = plsc.scan_count(ids)
            cur = plsc.load_gather(cur_v, [ids])
            pos = cur + (dup - cbase)
            plsc.store_scatter(stok_v, [pos], toks)
            plsc.store_scatter(sew_v, [pos], ewv)
            plsc.store_scatter(cur_v, [ids], pos + 1, mask=last)
            return 0
        lax.fori_loop(0, NCHUNK, place, 0)

        pltpu.sync_copy(stok_v, stok_hbm)
        pltpu.sync_copy(sew_v, sew_hbm)
        pltpu.sync_copy(off_v, off_hbm)
        pltpu.sync_copy(cnt_v, cnt_hbm)


def _sc_routing(ei, ew):
    mesh = plsc.VectorSubcoreMesh(core_axis_name="c", subcore_axis_name="s")
    fn = pl.kernel(
        _routing_body,
        mesh=mesh,
        compiler_params=pltpu.CompilerParams(needs_layout_passes=False),
        out_type=[
            jax.ShapeDtypeStruct((SLOTS_P,), jnp.int32),
            jax.ShapeDtypeStruct((SLOTS_P,), jnp.float32),
            jax.ShapeDtypeStruct((OFF_PAD,), jnp.int32),
            jax.ShapeDtypeStruct((NUM_EXPERTS,), jnp.int32),
        ],
        scratch_types=[
            pltpu.VMEM((SLOTS,), jnp.int32),
            pltpu.VMEM((SLOTS,), jnp.float32),
            pltpu.VMEM((SLOTS_P,), jnp.int32),
            pltpu.VMEM((SLOTS_P,), jnp.float32),
            pltpu.VMEM((OFF_PAD,), jnp.int32),
            pltpu.VMEM((NUM_EXPERTS,), jnp.int32),
            pltpu.VMEM((NUM_EXPERTS,), jnp.int32),
        ],
    )
    return fn(ei, ew)


def _moe_body(tok_ref, ew_ref, off_ref, cnt_ref, xs_ref, w1_ref, w2_ref,
              y_ref, eo_ref, o_ref):
    e = pl.program_id(0)

    @pl.when(e == 0)
    def _():
        y_ref[...] = jnp.zeros_like(y_ref)

    eo_ref[...] = jnp.zeros_like(eo_ref)

    start = pl.multiple_of(off_ref[e], 8)
    end = start + cnt_ref[e]
    nblk = (end - start + BLK - 1) // BLK

    def blk_body(b, _):
        base = start + b * BLK

        xg = xs_ref[pl.ds(base, BLK), :].astype(jnp.bfloat16)
        h = jax.nn.gelu(jnp.dot(xg, w1_ref[0].astype(jnp.bfloat16),
                                preferred_element_type=jnp.float32))
        o_ref[...] = jnp.dot(h.astype(jnp.bfloat16),
                             w2_ref[0].astype(jnp.bfloat16),
                             preferred_element_type=jnp.float32)

        rows = jnp.minimum(end - base, BLK)

        def scatter_row(r, _):
            slot = base + r
            tok = tok_ref[slot]
            row = o_ref[pl.ds(r, 1), :]
            y_ref[pl.ds(tok, 1), :] = y_ref[pl.ds(tok, 1), :] + row * ew_ref[slot]
            eo_ref[0, 0, pl.ds(tok, 1), :] = eo_ref[0, 0, pl.ds(tok, 1), :] + row
            return 0

        lax.fori_loop(0, rows, scatter_row, 0)
        return 0

    lax.fori_loop(0, nblk, blk_body, 0)


GCHUNK = SLOTS_P // 32


def _gather_body(stok_hbm, x_hbm, xs_hbm, idx_v, rows_v, sem):
    c = lax.axis_index("c")
    s = lax.axis_index("s")
    wid = s * 2 + c
    base = wid * GCHUNK
    pltpu.sync_copy(stok_hbm.at[pl.ds(base, GCHUNK)], idx_v)
    pltpu.async_copy(x_hbm.at[idx_v], rows_v, sem).wait()
    pltpu.sync_copy(rows_v, xs_hbm.at[pl.ds(base, GCHUNK)])


def _sc_gather(stok, xf):
    mesh = plsc.VectorSubcoreMesh(core_axis_name="c", subcore_axis_name="s")
    fn = pl.kernel(
        _gather_body,
        mesh=mesh,
        out_type=jax.ShapeDtypeStruct((SLOTS_P + BLK, D_MODEL), jnp.float32),
        scratch_types=[
            pltpu.VMEM((GCHUNK,), jnp.int32),
            pltpu.VMEM((GCHUNK, D_MODEL), jnp.float32),
            pltpu.SemaphoreType.DMA,
        ],
    )
    return fn(stok, xf)


@jax.jit
def _moe_call(sorted_tok, sorted_ew, offsets, counts, xs, w1, w2):
    grid_spec = pltpu.PrefetchScalarGridSpec(
        num_scalar_prefetch=4,
        grid=(NUM_EXPERTS,),
        in_specs=[
            pl.BlockSpec((SLOTS_P + BLK, D_MODEL), lambda e, *_: (0, 0)),
            pl.BlockSpec((1, D_MODEL, D_MODEL), lambda e, *_: (e, 0, 0)),
            pl.BlockSpec((1, D_MODEL, D_MODEL), lambda e, *_: (e, 0, 0)),
        ],
        out_specs=[
            pl.BlockSpec((SEQ, D_MODEL), lambda e, *_: (0, 0)),
            pl.BlockSpec((1, 1, SEQ, D_MODEL), lambda e, *_: (0, e, 0, 0)),
        ],
        scratch_shapes=[
            pltpu.VMEM((BLK, D_MODEL), jnp.float32),
        ],
    )
    return pl.pallas_call(
        _moe_body,
        grid_spec=grid_spec,
        out_shape=[
            jax.ShapeDtypeStruct((SEQ, D_MODEL), jnp.float32),
            jax.ShapeDtypeStruct((1, NUM_EXPERTS, SEQ, D_MODEL), jnp.float32),
        ],
    )(sorted_tok, sorted_ew, offsets, counts, xs, w1, w2)


def kernel(x, expert_weights, expert_indices, w1, w2):
    sl, bs, hs = x.shape
    xf = x.reshape(-1, hs)

    ei = expert_indices.reshape(-1).astype(jnp.int32)
    ewf = expert_weights.reshape(-1)
    sorted_tok, sorted_ew, offsets, counts = _sc_routing(ei, ewf)
    offsets = offsets[:NUM_EXPERTS + 1]
    xs = _sc_gather(sorted_tok, xf)

    y, eo = _moe_call(sorted_tok, sorted_ew, offsets, counts, xs, w1, w2)
    return y.reshape(sl, bs, hs), eo


# speedup vs baseline: 1.4192x; 1.1087x over previous
import jax
import jax.numpy as jnp
from jax import lax
from jax.experimental import pallas as pl
from jax.experimental.pallas import tpu as pltpu
from jax.experimental.pallas import tpu_sc as plsc

NUM_EXPERTS = 64
TOP_K = 2
D_MODEL = 768
SEQ = 2048
SLOTS = SEQ * TOP_K
BLK = 64
OFF_PAD = 80
LANES = 16
NCHUNK = SLOTS // LANES


def _routing_body(ei_hbm, ew_hbm, stok_hbm, sew_hbm, off_hbm,
                  ei_v, ew_v, stok_v, sew_v, off_v, cnt_v, cur_v):
    c = lax.axis_index("c")
    s = lax.axis_index("s")

    @pl.when(jnp.logical_and(c == 0, s == 0))
    def _():
        pltpu.sync_copy(ei_hbm, ei_v)
        pltpu.sync_copy(ew_hbm, ew_v)

        d0, _unused = plsc.scan_count(jnp.zeros((LANES,), jnp.int32))
        cbase = d0[0]

        zeros16 = jnp.zeros((LANES,), jnp.int32)
        for k in range(NUM_EXPERTS // LANES):
            cnt_v[pl.ds(k * LANES, LANES)] = zeros16

        def hist(i, _):
            ids = ei_v[pl.ds(i * LANES, LANES)]
            dup, last = plsc.scan_count(ids)
            plsc.addupdate_scatter(cnt_v, [ids], dup - cbase + 1, mask=last)
            return 0
        lax.fori_loop(0, NCHUNK, hist, 0)

        iota = lax.iota(jnp.int32, LANES)
        carry = jnp.int32(0)
        for k in range(NUM_EXPERTS // LANES):
            cnt = cnt_v[pl.ds(k * LANES, LANES)]
            inc = plsc.cumsum(cnt)
            excl = inc - cnt + carry
            cur_v[pl.ds(k * LANES, LANES)] = excl
            off_v[pl.ds(k * LANES, LANES)] = excl
            carry = carry + inc[LANES - 1]
        off_v[pl.ds(NUM_EXPERTS, LANES)] = jnp.where(iota == 0, carry, 0)

        def place(i, _):
            base = i * LANES
            ids = ei_v[pl.ds(base, LANES)]
            ewv = ew_v[pl.ds(base, LANES)]
            toks = lax.shift_right_logical(base + iota, 1)
            dup, last = plsc.scan_count(ids)
            cur = plsc.load_gather(cur_v, [ids])
            pos = cur + (dup - cbase)
            plsc.store_scatter(stok_v, [pos], toks)
            plsc.store_scatter(sew_v, [pos], ewv)
            plsc.store_scatter(cur_v, [ids], pos + 1, mask=last)
            return 0
        lax.fori_loop(0, NCHUNK, place, 0)

        pltpu.sync_copy(stok_v, stok_hbm)
        pltpu.sync_copy(sew_v, sew_hbm)
        pltpu.sync_copy(off_v, off_hbm)


def _sc_routing(ei, ew):
    mesh = plsc.VectorSubcoreMesh(core_axis_name="c", subcore_axis_name="s")
    fn = pl.kernel(
        _routing_body,
        mesh=mesh,
        compiler_params=pltpu.CompilerParams(needs_layout_passes=False),
        out_type=[
            jax.ShapeDtypeStruct((SLOTS,), jnp.int32),
            jax.ShapeDtypeStruct((SLOTS,), jnp.float32),
            jax.ShapeDtypeStruct((OFF_PAD,), jnp.int32),
        ],
        scratch_types=[
            pltpu.VMEM((SLOTS,), jnp.int32),
            pltpu.VMEM((SLOTS,), jnp.float32),
            pltpu.VMEM((SLOTS,), jnp.int32),
            pltpu.VMEM((SLOTS,), jnp.float32),
            pltpu.VMEM((OFF_PAD,), jnp.int32),
            pltpu.VMEM((NUM_EXPERTS,), jnp.int32),
            pltpu.VMEM((NUM_EXPERTS,), jnp.int32),
        ],
    )
    return fn(ei, ew)


def _moe_body(tok_ref, ew_ref, off_ref, x_ref, w1_ref, w2_ref, y_ref, eo_ref,
              xg_ref, o_ref):
    e = pl.program_id(0)

    @pl.when(e == 0)
    def _():
        y_ref[...] = jnp.zeros_like(y_ref)

    eo_ref[...] = jnp.zeros_like(eo_ref)

    start = off_ref[e]
    end = off_ref[e + 1]
    nblk = (end - start + BLK - 1) // BLK

    def blk_body(b, _):
        base = start + b * BLK
        rows = jnp.minimum(end - base, BLK)

        def gather_row(r, _):
            tok = tok_ref[base + r]
            xg_ref[pl.ds(r, 1), :] = x_ref[pl.ds(tok, 1), :]
            return 0

        lax.fori_loop(0, rows, gather_row, 0)

        h = jax.nn.gelu(jnp.dot(xg_ref[...], w1_ref[0],
                                preferred_element_type=jnp.float32))
        o_ref[...] = jnp.dot(h, w2_ref[0], preferred_element_type=jnp.float32)

        def scatter_row(r, _):
            slot = base + r
            tok = tok_ref[slot]
            row = o_ref[pl.ds(r, 1), :]
            y_ref[pl.ds(tok, 1), :] = y_ref[pl.ds(tok, 1), :] + row * ew_ref[slot]
            eo_ref[0, 0, pl.ds(tok, 1), :] = eo_ref[0, 0, pl.ds(tok, 1), :] + row
            return 0

        lax.fori_loop(0, rows, scatter_row, 0)
        return 0

    lax.fori_loop(0, nblk, blk_body, 0)


@jax.jit
def _moe_call(sorted_tok, sorted_ew, offsets, xf, w1, w2):
    grid_spec = pltpu.PrefetchScalarGridSpec(
        num_scalar_prefetch=3,
        grid=(NUM_EXPERTS,),
        in_specs=[
            pl.BlockSpec((SEQ, D_MODEL), lambda e, *_: (0, 0)),
            pl.BlockSpec((1, D_MODEL, D_MODEL), lambda e, *_: (e, 0, 0)),
            pl.BlockSpec((1, D_MODEL, D_MODEL), lambda e, *_: (e, 0, 0)),
        ],
        out_specs=[
            pl.BlockSpec((SEQ, D_MODEL), lambda e, *_: (0, 0)),
            pl.BlockSpec((1, 1, SEQ, D_MODEL), lambda e, *_: (0, e, 0, 0)),
        ],
        scratch_shapes=[
            pltpu.VMEM((BLK, D_MODEL), jnp.float32),
            pltpu.VMEM((BLK, D_MODEL), jnp.float32),
        ],
    )
    return pl.pallas_call(
        _moe_body,
        grid_spec=grid_spec,
        out_shape=[
            jax.ShapeDtypeStruct((SEQ, D_MODEL), jnp.float32),
            jax.ShapeDtypeStruct((1, NUM_EXPERTS, SEQ, D_MODEL), jnp.float32),
        ],
    )(sorted_tok, sorted_ew, offsets, xf, w1, w2)


def kernel(x, expert_weights, expert_indices, w1, w2):
    sl, bs, hs = x.shape
    xf = x.reshape(-1, hs)

    ei = expert_indices.reshape(-1).astype(jnp.int32)
    ewf = expert_weights.reshape(-1)
    sorted_tok, sorted_ew, offsets = _sc_routing(ei, ewf)

    y, eo = _moe_call(sorted_tok, sorted_ew, offsets, xf, w1, w2)
    return y.reshape(sl, bs, hs), eo


# speedup vs baseline: 1.4224x; 1.0023x over previous
import jax
import jax.numpy as jnp
from jax import lax
from jax.experimental import pallas as pl
from jax.experimental.pallas import tpu as pltpu
from jax.experimental.pallas import tpu_sc as plsc

NUM_EXPERTS = 64
TOP_K = 2
D_MODEL = 768
SEQ = 2048
SLOTS = SEQ * TOP_K
BLK = 64
OFF_PAD = 80
LANES = 16
NCHUNK = SLOTS // LANES
PK_LEN = 2 * SLOTS + OFF_PAD


def _routing_body(ei_hbm, ew_hbm, pk_hbm, ei_v, ew_v, pk_v, cnt_v, cur_v):
    c = lax.axis_index("c")
    s = lax.axis_index("s")

    @pl.when(jnp.logical_and(c == 0, s == 0))
    def _():
        pltpu.sync_copy(ei_hbm, ei_v)
        pltpu.sync_copy(ew_hbm, ew_v)

        d0, _unused = plsc.scan_count(jnp.zeros((LANES,), jnp.int32))
        cbase = d0[0]

        zeros16 = jnp.zeros((LANES,), jnp.int32)
        for k in range(NUM_EXPERTS // LANES):
            cnt_v[pl.ds(k * LANES, LANES)] = zeros16

        def hist(i, _):
            ids = ei_v[pl.ds(i * LANES, LANES)]
            dup, last = plsc.scan_count(ids)
            plsc.addupdate_scatter(cnt_v, [ids], dup - cbase + 1, mask=last)
            return 0
        lax.fori_loop(0, NCHUNK, hist, 0)

        iota = lax.iota(jnp.int32, LANES)
        carry = jnp.int32(0)
        for k in range(NUM_EXPERTS // LANES):
            cnt = cnt_v[pl.ds(k * LANES, LANES)]
            inc = plsc.cumsum(cnt)
            excl = inc - cnt + carry
            cur_v[pl.ds(k * LANES, LANES)] = excl
            pk_v[pl.ds(2 * SLOTS + k * LANES, LANES)] = excl
            carry = carry + inc[LANES - 1]
        pk_v[pl.ds(2 * SLOTS + NUM_EXPERTS, LANES)] = jnp.where(
            iota == 0, carry, 0)

        def place(i, _):
            base = i * LANES
            ids = ei_v[pl.ds(base, LANES)]
            ewv = ew_v[pl.ds(base, LANES)]
            toks = lax.shift_right_logical(base + iota, 1)
            dup, last = plsc.scan_count(ids)
            cur = plsc.load_gather(cur_v, [ids])
            pos = cur + (dup - cbase)
            plsc.store_scatter(pk_v, [pos], toks)
            plsc.store_scatter(pk_v, [pos + SLOTS],
                               plsc.bitcast(ewv, jnp.int32))
            plsc.store_scatter(cur_v, [ids], pos + 1, mask=last)
            return 0
        lax.fori_loop(0, NCHUNK, place, 0)

        pltpu.sync_copy(pk_v, pk_hbm)


def _sc_routing(ei, ew):
    mesh = plsc.VectorSubcoreMesh(core_axis_name="c", subcore_axis_name="s")
    fn = pl.kernel(
        _routing_body,
        mesh=mesh,
        compiler_params=pltpu.CompilerParams(needs_layout_passes=False),
        out_type=jax.ShapeDtypeStruct((PK_LEN,), jnp.int32),
        scratch_types=[
            pltpu.VMEM((SLOTS,), jnp.int32),
            pltpu.VMEM((SLOTS,), jnp.float32),
            pltpu.VMEM((PK_LEN,), jnp.int32),
            pltpu.VMEM((NUM_EXPERTS,), jnp.int32),
            pltpu.VMEM((NUM_EXPERTS,), jnp.int32),
        ],
    )
    return fn(ei, ew)


def _moe_body(pk_ref, x_ref, w1_ref, w2_ref, y_ref, eo_ref, xg_ref, o_ref):
    e = pl.program_id(0)

    @pl.when(e == 0)
    def _():
        y_ref[...] = jnp.zeros_like(y_ref)

    eo_ref[...] = jnp.zeros_like(eo_ref)

    start = pk_ref[2 * SLOTS + e]
    end = pk_ref[2 * SLOTS + e + 1]
    nblk = (end - start + BLK - 1) // BLK

    def blk_body(b, _):
        base = start + b * BLK
        rows = jnp.minimum(end - base, BLK)

        def gather_row(r, _):
            tok = pk_ref[base + r]
            xg_ref[pl.ds(r, 1), :] = x_ref[pl.ds(tok, 1), :]
            return 0

        lax.fori_loop(0, rows, gather_row, 0)

        h = jax.nn.gelu(jnp.dot(xg_ref[...], w1_ref[0],
                                preferred_element_type=jnp.float32))
        o_ref[...] = jnp.dot(h, w2_ref[0], preferred_element_type=jnp.float32)

        def scatter_row(r, _):
            slot = base + r
            tok = pk_ref[slot]
            ew = lax.bitcast_convert_type(pk_ref[SLOTS + slot], jnp.float32)
            row = o_ref[pl.ds(r, 1), :]
            y_ref[pl.ds(tok, 1), :] = y_ref[pl.ds(tok, 1), :] + row * ew
            eo_ref[0, 0, pl.ds(tok, 1), :] = eo_ref[0, 0, pl.ds(tok, 1), :] + row
            return 0

        lax.fori_loop(0, rows, scatter_row, 0)
        return 0

    lax.fori_loop(0, nblk, blk_body, 0)


@jax.jit
def _moe_call(pk, xf, w1, w2):
    grid_spec = pltpu.PrefetchScalarGridSpec(
        num_scalar_prefetch=1,
        grid=(NUM_EXPERTS,),
        in_specs=[
            pl.BlockSpec((SEQ, D_MODEL), lambda e, *_: (0, 0)),
            pl.BlockSpec((1, D_MODEL, D_MODEL), lambda e, *_: (e, 0, 0)),
            pl.BlockSpec((1, D_MODEL, D_MODEL), lambda e, *_: (e, 0, 0)),
        ],
        out_specs=[
            pl.BlockSpec((SEQ, D_MODEL), lambda e, *_: (0, 0)),
            pl.BlockSpec((1, 1, SEQ, D_MODEL), lambda e, *_: (0, e, 0, 0)),
        ],
        scratch_shapes=[
            pltpu.VMEM((BLK, D_MODEL), jnp.float32),
            pltpu.VMEM((BLK, D_MODEL), jnp.float32),
        ],
    )
    return pl.pallas_call(
        _moe_body,
        grid_spec=grid_spec,
        out_shape=[
            jax.ShapeDtypeStruct((SEQ, D_MODEL), jnp.float32),
            jax.ShapeDtypeStruct((1, NUM_EXPERTS, SEQ, D_MODEL), jnp.float32),
        ],
    )(pk, xf, w1, w2)


def kernel(x, expert_weights, expert_indices, w1, w2):
    sl, bs, hs = x.shape
    xf = x.reshape(-1, hs)

    ei = expert_indices.reshape(-1).astype(jnp.int32)
    ewf = expert_weights.reshape(-1)
    pk = _sc_routing(ei, ewf)

    y, eo = _moe_call(pk, xf, w1, w2)
    return y.reshape(sl, bs, hs), eo
